# Initial kernel scaffold; baseline (speedup 1.0000x reference)
import jax
import jax.numpy as jnp
from jax.experimental import pallas as pl


def kernel(x, edge_index, edge_attr, batch, W_l, b_l, W_r, b_r, W_e, att, bias):
    raise NotImplementedError("write your pallas kernel here")


# speedup vs baseline: 13.3232x; 13.3232x over previous
import functools

import jax
import jax.numpy as jnp
from jax import lax
from jax.experimental import pallas as pl
from jax.experimental.pallas import tpu as pltpu
from jax.experimental.pallas import tpu_sc as plsc

N = 10000
E = 320000
F_IN = 128
C = 32
G = 64

K = 128


def _hsum16(v):
    i = lax.broadcasted_iota(jnp.int32, (16,), 0)
    dnums = lax.GatherDimensionNumbers(
        offset_dims=(), collapsed_slice_dims=(0,), start_index_map=(0,))
    for sh in (8, 4, 2, 1):
        idx = jnp.bitwise_and(i + sh, 15)
        v = v + lax.gather(v, idx[:, None], dnums, (1,),
                           mode=lax.GatherScatterMode.PROMISE_IN_BOUNDS)
    return v
ROW = 48
NR = N + 112


def _mm_kernel(x_ref, w_ref, b_ref, ea_ref, xl_ref, xr_ref, m_ref):
    y = jnp.dot(x_ref[...], w_ref[...], preferred_element_type=jnp.float32)
    y = y + b_ref[...]
    xl_ref[...] = y[:, :C]
    xr_ref[...] = y[:, C:]
    m_ref[...] = jnp.sum(ea_ref[...]).reshape(1, 1) * (1.0 / E)


def _final_kernel(p0_ref, p1_ref, bias_ref, batch_ref, out_ref):
    acc = p0_ref[...] + p1_ref[...]
    num = acc[:N, :C]
    den = acc[:N, C:C + 1]
    node = num / (den + 1e-16) + bias_ref[...]
    b = batch_ref[...]
    gi = lax.broadcasted_iota(jnp.int32, (G, N), 0)
    oneh = (gi == b).astype(jnp.float32)
    sums = jnp.dot(oneh, node, preferred_element_type=jnp.float32)
    counts = jnp.sum(oneh, axis=1)
    out_ref[...] = sums / jnp.maximum(counts, 1.0)[:, None]


def _make_sc_kernel(num_cores, num_subcores, epw):
    n_chunks = epw // K
    rows_per_tile = NR // num_subcores
    mesh = plsc.VectorSubcoreMesh(core_axis_name="c", subcore_axis_name="s")

    @functools.partial(
        pl.kernel,
        mesh=mesh,
        out_type=jax.ShapeDtypeStruct((num_cores, NR, ROW), jnp.float32),
        compiler_params=pltpu.CompilerParams(use_tc_tiling_on_sc=False),
        scratch_types=[
            pltpu.VMEM((K,), jnp.int32),
            pltpu.VMEM((K,), jnp.int32),
            pltpu.VMEM((K,), jnp.float32),
            pltpu.VMEM((K, C), jnp.float32),
            pltpu.VMEM((K, C), jnp.float32),
            pltpu.VMEM((K, ROW), jnp.float32),
            pltpu.VMEM((C,), jnp.float32),
            pltpu.VMEM((C,), jnp.float32),
            pltpu.VMEM_SHARED((NR, ROW), jnp.float32),
            pltpu.SemaphoreType.DMA,
            pltpu.SemaphoreType.DMA,
        ],
    )
    def sc_kernel(xl_hbm, xr_hbm, src_hbm, dst_hbm, ea_hbm, we_hbm, att_hbm,
                  zero_hbm, out_hbm,
                  srcv, dstv, eav, xlr, xrr, sbuf, wev, attv, acc, sem1, sem2):
        cid = lax.axis_index("c")
        sid = lax.axis_index("s")
        wid = sid * num_cores + cid
        base = wid * epw

        r0 = sid * rows_per_tile
        pltpu.sync_copy(zero_hbm.at[pl.ds(r0, rows_per_tile)],
                        acc.at[pl.ds(r0, rows_per_tile)])
        pltpu.sync_copy(we_hbm, wev)
        pltpu.sync_copy(att_hbm, attv)
        plsc.subcore_barrier()

        we0 = wev[pl.ds(0, 16)]
        we1 = wev[pl.ds(16, 16)]
        att0 = attv[pl.ds(0, 16)]
        att1 = attv[pl.ds(16, 16)]
        lane0 = lax.broadcasted_iota(jnp.int32, (16,), 0) == 0

        def granule_body(g, carry):
            eag = eav[pl.ds(g * 16, 16)]
            for i in range(16):
                j = g * 16 + i
                xl0 = xlr[j, pl.ds(0, 16)]
                xl1 = xlr[j, pl.ds(16, 16)]
                xr0 = xrr[j, pl.ds(0, 16)]
                xr1 = xrr[j, pl.ds(16, 16)]
                ea = eag[i]
                m0 = xl0 + xr0 + ea * we0
                m1 = xl1 + xr1 + ea * we1
                m0 = jnp.where(m0 >= 0.0, m0, m0 * 0.2)
                m1 = jnp.where(m1 >= 0.0, m1, m1 * 0.2)
                av = jnp.exp(_hsum16(m0 * att0 + m1 * att1))
                sbuf[j, pl.ds(0, 16)] = xl0 * av
                sbuf[j, pl.ds(16, 16)] = xl1 * av
                sbuf[j, pl.ds(32, 16)] = jnp.where(lane0, av, 0.0)
            return carry

        def chunk_body(ci, carry):
            off = base + ci * K
            pltpu.sync_copy(src_hbm.at[pl.ds(off, K)], srcv)
            pltpu.sync_copy(dst_hbm.at[pl.ds(off, K)], dstv)
            pltpu.sync_copy(ea_hbm.at[pl.ds(off, K)], eav)
            g1 = pltpu.async_copy(xl_hbm.at[srcv], xlr, sem1)
            g2 = pltpu.async_copy(xr_hbm.at[dstv], xrr, sem2)
            g1.wait()
            g2.wait()
            lax.fori_loop(0, K // 16, granule_body, carry)
            pltpu.sync_copy(sbuf, acc.at[dstv], add=True)
            return carry

        lax.fori_loop(0, n_chunks, chunk_body, jnp.int32(0))
        plsc.subcore_barrier()
        pltpu.sync_copy(acc.at[pl.ds(r0, rows_per_tile)],
                        out_hbm.at[cid, pl.ds(r0, rows_per_tile)])

    return sc_kernel


def kernel(x, edge_index, edge_attr, batch, W_l, b_l, W_r, b_r, W_e, att, bias):
    info = plsc.get_sparse_core_info()
    num_cores, num_subcores = info.num_cores, info.num_subcores
    nw = num_cores * num_subcores

    w2 = jnp.concatenate([W_l, W_r], axis=1)
    b2 = jnp.concatenate([b_l, b_r])[None, :]
    ea2 = edge_attr.reshape(2500, 128)
    x_l, x_r, ea_mean = pl.pallas_call(
        _mm_kernel,
        out_shape=(
            jax.ShapeDtypeStruct((N, C), jnp.float32),
            jax.ShapeDtypeStruct((N, C), jnp.float32),
            jax.ShapeDtypeStruct((1, 1), jnp.float32),
        ),
    )(x, w2, b2, ea2)

    e_tot = E + N
    epad = ((e_tot + nw * K - 1) // (nw * K)) * (nw * K)
    loop = jnp.arange(N, dtype=jnp.int32)
    pad = epad - e_tot
    src = jnp.concatenate([edge_index[0], loop,
                           jnp.zeros((pad,), jnp.int32)])
    dst = jnp.concatenate([edge_index[1], loop,
                           jnp.full((pad,), N, jnp.int32)])
    ea = jnp.concatenate([edge_attr[:, 0],
                          jnp.broadcast_to(ea_mean[0, 0], (N,)),
                          jnp.zeros((pad,), jnp.float32)])

    sc = _make_sc_kernel(num_cores, num_subcores, epad // nw)
    parts = sc(x_l, x_r, src, dst, ea, W_e[0], att,
               jnp.zeros((NR, ROW), jnp.float32))

    p0 = parts[0]
    p1 = parts[1] if num_cores > 1 else jnp.zeros_like(parts[0])
    pooled = pl.pallas_call(
        _final_kernel,
        out_shape=jax.ShapeDtypeStruct((G, C), jnp.float32),
    )(p0, p1, bias[None, :], batch[None, :].astype(jnp.int32))
    return pooled


# speedup vs baseline: 14.8216x; 1.1125x over previous
import functools

import jax
import jax.numpy as jnp
from jax import lax
from jax.experimental import pallas as pl
from jax.experimental.pallas import tpu as pltpu
from jax.experimental.pallas import tpu_sc as plsc

N = 10000
E = 320000
F_IN = 128
C = 32
G = 64

K = 128


def _hsum16(v):
    i = lax.broadcasted_iota(jnp.int32, (16,), 0)
    dnums = lax.GatherDimensionNumbers(
        offset_dims=(), collapsed_slice_dims=(0,), start_index_map=(0,))
    for sh in (8, 4, 2, 1):
        idx = jnp.bitwise_and(i + sh, 15)
        v = v + lax.gather(v, idx[:, None], dnums, (1,),
                           mode=lax.GatherScatterMode.PROMISE_IN_BOUNDS)
    return v
ROW = 48
NR = N + 112


def _mm_kernel(x_ref, w_ref, b_ref, ea_ref, xl_ref, xr_ref, m_ref):
    y = jnp.dot(x_ref[...], w_ref[...], preferred_element_type=jnp.float32)
    y = y + b_ref[...]
    xl_ref[...] = y[:, :C]
    xr_ref[...] = y[:, C:]
    m_ref[...] = jnp.sum(ea_ref[...]).reshape(1, 1) * (1.0 / E)


def _final_kernel(p0_ref, p1_ref, bias_ref, batch_ref, out_ref):
    acc = p0_ref[...] + p1_ref[...]
    num = acc[:N, :C]
    den = acc[:N, C:C + 1]
    node = num / (den + 1e-16) + bias_ref[...]
    b = batch_ref[...]
    gi = lax.broadcasted_iota(jnp.int32, (G, N), 0)
    oneh = (gi == b).astype(jnp.float32)
    sums = jnp.dot(oneh, node, preferred_element_type=jnp.float32)
    counts = jnp.sum(oneh, axis=1)
    out_ref[...] = sums / jnp.maximum(counts, 1.0)[:, None]


def _make_sc_kernel(num_cores, num_subcores, epw):
    n_chunks = epw // K
    rows_per_tile = NR // num_subcores
    mesh = plsc.VectorSubcoreMesh(core_axis_name="c", subcore_axis_name="s")

    @functools.partial(
        pl.kernel,
        mesh=mesh,
        out_type=jax.ShapeDtypeStruct((num_cores, NR, ROW), jnp.float32),
        compiler_params=pltpu.CompilerParams(use_tc_tiling_on_sc=False),
        scratch_types=[
            pltpu.VMEM((K,), jnp.int32),
            pltpu.VMEM((K,), jnp.int32),
            pltpu.VMEM((K,), jnp.float32),
            pltpu.VMEM((K, C), jnp.float32),
            pltpu.VMEM((K, C), jnp.float32),
            pltpu.VMEM((K,), jnp.int32),
            pltpu.VMEM((K,), jnp.int32),
            pltpu.VMEM((K,), jnp.float32),
            pltpu.VMEM((K, C), jnp.float32),
            pltpu.VMEM((K, C), jnp.float32),
            pltpu.VMEM((K, ROW), jnp.float32),
            pltpu.VMEM((C,), jnp.float32),
            pltpu.VMEM((C,), jnp.float32),
            pltpu.VMEM_SHARED((NR, ROW), jnp.float32),
            pltpu.SemaphoreType.DMA,
            pltpu.SemaphoreType.DMA,
            pltpu.SemaphoreType.DMA,
            pltpu.SemaphoreType.DMA,
        ],
    )
    def sc_kernel(xl_hbm, xr_hbm, src_hbm, dst_hbm, ea_hbm, we_hbm, att_hbm,
                  zero_hbm, out_hbm,
                  srcv0, dstv0, eav0, xlr0, xrr0,
                  srcv1, dstv1, eav1, xlr1, xrr1,
                  sbuf, wev, attv, acc, semA0, semB0, semA1, semB1):
        cid = lax.axis_index("c")
        sid = lax.axis_index("s")
        wid = sid * num_cores + cid
        base = wid * epw

        r0 = sid * rows_per_tile
        pltpu.sync_copy(zero_hbm.at[pl.ds(r0, rows_per_tile)],
                        acc.at[pl.ds(r0, rows_per_tile)])
        pltpu.sync_copy(we_hbm, wev)
        pltpu.sync_copy(att_hbm, attv)
        plsc.subcore_barrier()

        we0 = wev[pl.ds(0, 16)]
        we1 = wev[pl.ds(16, 16)]
        att0 = attv[pl.ds(0, 16)]
        att1 = attv[pl.ds(16, 16)]
        lane0 = lax.broadcasted_iota(jnp.int32, (16,), 0) == 0

        def fire(c, srcv, dstv, eav, xlr, xrr, semA, semB):
            off = base + c * K
            pltpu.sync_copy(src_hbm.at[pl.ds(off, K)], srcv)
            pltpu.sync_copy(dst_hbm.at[pl.ds(off, K)], dstv)
            pltpu.sync_copy(ea_hbm.at[pl.ds(off, K)], eav)
            pltpu.async_copy(xl_hbm.at[srcv], xlr, semA)
            pltpu.async_copy(xr_hbm.at[dstv], xrr, semB)

        def waitg(srcv, dstv, xlr, xrr, semA, semB):
            pltpu.make_async_copy(xl_hbm.at[srcv], xlr, semA).wait()
            pltpu.make_async_copy(xr_hbm.at[dstv], xrr, semB).wait()

        def compute_chunk(eav, xlr, xrr, dstv):
            def granule_body(g, carry):
                eag = eav[pl.ds(g * 16, 16)]
                for i in range(16):
                    j = g * 16 + i
                    xl0 = xlr[j, pl.ds(0, 16)]
                    xl1 = xlr[j, pl.ds(16, 16)]
                    xr0 = xrr[j, pl.ds(0, 16)]
                    xr1 = xrr[j, pl.ds(16, 16)]
                    ea = eag[i]
                    m0 = xl0 + xr0 + ea * we0
                    m1 = xl1 + xr1 + ea * we1
                    m0 = jnp.where(m0 >= 0.0, m0, m0 * 0.2)
                    m1 = jnp.where(m1 >= 0.0, m1, m1 * 0.2)
                    av = jnp.exp(_hsum16(m0 * att0 + m1 * att1))
                    sbuf[j, pl.ds(0, 16)] = xl0 * av
                    sbuf[j, pl.ds(16, 16)] = xl1 * av
                    sbuf[j, pl.ds(32, 16)] = jnp.where(lane0, av, 0.0)
                return carry

            lax.fori_loop(0, K // 16, granule_body, jnp.int32(0))
            pltpu.sync_copy(sbuf, acc.at[dstv], add=True)

        n_pairs = n_chunks // 2
        fire(jnp.int32(0), srcv0, dstv0, eav0, xlr0, xrr0, semA0, semB0)

        def pair_body(p, carry):
            c0 = p * 2
            fire(c0 + 1, srcv1, dstv1, eav1, xlr1, xrr1, semA1, semB1)
            waitg(srcv0, dstv0, xlr0, xrr0, semA0, semB0)
            compute_chunk(eav0, xlr0, xrr0, dstv0)

            @pl.when(p < n_pairs - 1)
            def _():
                fire(c0 + 2, srcv0, dstv0, eav0, xlr0, xrr0, semA0, semB0)

            waitg(srcv1, dstv1, xlr1, xrr1, semA1, semB1)
            compute_chunk(eav1, xlr1, xrr1, dstv1)
            return carry

        lax.fori_loop(0, n_pairs, pair_body, jnp.int32(0))
        plsc.subcore_barrier()
        pltpu.sync_copy(acc.at[pl.ds(r0, rows_per_tile)],
                        out_hbm.at[cid, pl.ds(r0, rows_per_tile)])

    return sc_kernel


def kernel(x, edge_index, edge_attr, batch, W_l, b_l, W_r, b_r, W_e, att, bias):
    info = plsc.get_sparse_core_info()
    num_cores, num_subcores = info.num_cores, info.num_subcores
    nw = num_cores * num_subcores

    w2 = jnp.concatenate([W_l, W_r], axis=1)
    b2 = jnp.concatenate([b_l, b_r])[None, :]
    ea2 = edge_attr.reshape(2500, 128)
    x_l, x_r, ea_mean = pl.pallas_call(
        _mm_kernel,
        out_shape=(
            jax.ShapeDtypeStruct((N, C), jnp.float32),
            jax.ShapeDtypeStruct((N, C), jnp.float32),
            jax.ShapeDtypeStruct((1, 1), jnp.float32),
        ),
    )(x, w2, b2, ea2)

    e_tot = E + N
    epad = ((e_tot + 2 * nw * K - 1) // (2 * nw * K)) * (2 * nw * K)
    loop = jnp.arange(N, dtype=jnp.int32)
    pad = epad - e_tot
    src = jnp.concatenate([edge_index[0], loop,
                           jnp.zeros((pad,), jnp.int32)])
    dst = jnp.concatenate([edge_index[1], loop,
                           jnp.full((pad,), N, jnp.int32)])
    ea = jnp.concatenate([edge_attr[:, 0],
                          jnp.broadcast_to(ea_mean[0, 0], (N,)),
                          jnp.zeros((pad,), jnp.float32)])

    sc = _make_sc_kernel(num_cores, num_subcores, epad // nw)
    parts = sc(x_l, x_r, src, dst, ea, W_e[0], att,
               jnp.zeros((NR, ROW), jnp.float32))

    p0 = parts[0]
    p1 = parts[1] if num_cores > 1 else jnp.zeros_like(parts[0])
    pooled = pl.pallas_call(
        _final_kernel,
        out_shape=jax.ShapeDtypeStruct((G, C), jnp.float32),
    )(p0, p1, bias[None, :], batch[None, :].astype(jnp.int32))
    return pooled
